# baseline (speedup 1.0000x reference)
import functools

import jax
import jax.numpy as jnp
from jax import lax
from jax.experimental import pallas as pl
from jax.experimental.pallas import tpu as pltpu
from jax.experimental.pallas import tpu_sc as plsc

NUM_CLASSES = 1000000
HIDDEN = 64
BATCH = 16384

CH = 128


@functools.lru_cache(maxsize=None)
def _build(batch, hidden):
    info = plsc.get_sparse_core_info()
    nw = info.num_cores * info.num_subcores
    bpw = batch // nw
    nc = info.num_cores
    L = info.num_lanes
    h2 = 2 * hidden

    mesh = plsc.VectorSubcoreMesh(core_axis_name="c", subcore_axis_name="s")

    @functools.partial(
        pl.kernel,
        mesh=mesh,
        out_type=jax.ShapeDtypeStruct((batch, hidden), jnp.float32),
        scratch_types=[
            pltpu.VMEM((bpw,), jnp.int32),
            pltpu.VMEM((bpw + L,), jnp.int32),
            pltpu.VMEM((CH, h2), jnp.float32),
            pltpu.VMEM((CH, hidden), jnp.float32),
            pltpu.SemaphoreType.DMA,
        ],
    )
    def k(idx2_hbm, par_hbm, table2_hbm, out_hbm,
          idx2_v, par_v, rows2_v, out_v, sem):
        t = lax.axis_index("s") * nc + lax.axis_index("c")
        base = t * bpw
        pltpu.sync_copy(idx2_hbm.at[pl.ds(base, bpw)], idx2_v)
        pltpu.sync_copy(par_hbm.at[pl.ds(base, bpw)], par_v.at[pl.ds(0, bpw)])

        for ch in range(bpw // CH):
            pltpu.async_copy(
                table2_hbm.at[idx2_v.at[pl.ds(ch * CH, CH)]], rows2_v, sem
            ).wait()

            def body(r, _):
                p = par_v[pl.ds(ch * CH + r, L)][0]
                off = p * hidden
                for q in range(hidden // L):
                    out_v[r, pl.ds(q * L, L)] = \
                        rows2_v[r, pl.ds(off + q * L, L)]
                return 0

            lax.fori_loop(0, CH, body, 0)
            pltpu.sync_copy(out_v, out_hbm.at[pl.ds(base + ch * CH, CH)])

    return k


def kernel(labels, embedding_table):
    idx = labels.astype(jnp.int32)
    table2 = embedding_table.reshape(-1, 2 * embedding_table.shape[1])
    return _build(idx.shape[0], embedding_table.shape[1])(
        idx >> 1, idx & 1, table2)


# speedup vs baseline: 1.0235x; 1.0235x over previous
import functools

import jax
import jax.numpy as jnp
from jax import lax
from jax.experimental import pallas as pl
from jax.experimental.pallas import tpu as pltpu
from jax.experimental.pallas import tpu_sc as plsc

NUM_CLASSES = 1000000
HIDDEN = 64
BATCH = 16384


@functools.lru_cache(maxsize=None)
def _build(batch, hidden):
    info = plsc.get_sparse_core_info()
    nw = info.num_cores * info.num_subcores
    bpw = batch // nw
    nc = info.num_cores

    mesh = plsc.VectorSubcoreMesh(core_axis_name="c", subcore_axis_name="s")

    @functools.partial(
        pl.kernel,
        mesh=mesh,
        compiler_params=pltpu.CompilerParams(use_tc_tiling_on_sc=False),
        out_type=jax.ShapeDtypeStruct((batch, hidden), jnp.float32),
        scratch_types=[
            pltpu.VMEM((bpw,), jnp.int32),
            pltpu.VMEM((bpw, hidden), jnp.float32),
            pltpu.SemaphoreType.DMA,
        ],
    )
    def gather_kernel(idx_hbm, table_hbm, out_hbm, idx_v, rows_v, sem):
        wid = lax.axis_index("s") * nc + lax.axis_index("c")
        base = wid * bpw
        pltpu.sync_copy(idx_hbm.at[pl.ds(base, bpw)], idx_v)
        pltpu.async_copy(table_hbm.at[idx_v], rows_v, sem).wait()
        pltpu.sync_copy(rows_v, out_hbm.at[pl.ds(base, bpw)])

    return gather_kernel


def kernel(labels, embedding_table):
    idx = labels.astype(jnp.int32)
    return _build(idx.shape[0], embedding_table.shape[1])(idx, embedding_table)
